# Initial kernel scaffold; baseline (speedup 1.0000x reference)
import jax
import jax.numpy as jnp
from jax.experimental import pallas as pl


def kernel(x, edge_index, We1, be1, We2, be2, We3, be3, Wmu, bmu, Wlv, blv, Wd1, bd1, Wd2, bd2, Wd3, bd3):
    raise NotImplementedError("write your pallas kernel here")


# speedup vs baseline: 6.8025x; 6.8025x over previous
import functools

import jax
import jax.numpy as jnp
from jax import lax
from jax.experimental import pallas as pl
from jax.experimental.pallas import tpu as pltpu
from jax.experimental.pallas import tpu_sc as plsc

N = 10000
E = 320000
NPAD = 10240
EPAD = 327680
CHUNK = 128
TILES = 16
CPT = EPAD // TILES // CHUNK
CPT_DEG = EPAD // 2 // TILES // CHUNK
ROWS_T = NPAD // TILES
DPAD = 10016
BR = 2000
BCH = 16


def _mesh():
    return plsc.VectorSubcoreMesh(core_axis_name="c", subcore_axis_name="s")


@functools.cache
def _make_deg():
    @functools.partial(
        pl.kernel,
        out_type=jax.ShapeDtypeStruct((2, TILES, NPAD // TILES), jnp.float32),
        mesh=_mesh(),
        scratch_types=[
            pltpu.VMEM((CPT_DEG, CHUNK), jnp.int32),
            pltpu.VMEM((CHUNK,), jnp.float32),
            pltpu.VMEM((NPAD // TILES,), jnp.float32),
            pltpu.VMEM_SHARED((NPAD,), jnp.float32),
        ],
    )
    def degk(dst2d, out, didx, ones, zeros, acc):
        c = lax.axis_index("c")
        s = lax.axis_index("s")
        for i in range(CHUNK // 16):
            ones[pl.ds(i * 16, 16)] = jnp.full((16,), 1.0, jnp.float32)
        for i in range(NPAD // TILES // 16):
            zeros[pl.ds(i * 16, 16)] = jnp.zeros((16,), jnp.float32)
        pltpu.sync_copy(zeros, acc.at[pl.ds(s * (NPAD // TILES), NPAD // TILES)])
        plsc.subcore_barrier()
        pltpu.sync_copy(
            dst2d.at[pl.ds(c * (TILES * CPT_DEG) + s * CPT_DEG, CPT_DEG)], didx)

        def body(j, carry):
            pltpu.sync_copy(ones, acc.at[didx.at[j]], add=True)
            return carry

        lax.fori_loop(0, CPT_DEG, body, 0)
        plsc.subcore_barrier()
        pltpu.sync_copy(acc.at[pl.ds(s * (NPAD // TILES), NPAD // TILES)],
                        out.at[c, s])

    return degk


def _deg(dst2d):
    return _make_deg()(dst2d)


@functools.cache
def _make_prop_fs():
    Fh = 128

    @functools.partial(
        pl.kernel,
        out_type=jax.ShapeDtypeStruct((2, NPAD, Fh), jnp.float32),
        mesh=_mesh(),
        scratch_types=[
            pltpu.VMEM((BCH, CHUNK), jnp.int32),
            pltpu.VMEM((BCH, CHUNK), jnp.int32),
            pltpu.VMEM((CHUNK, Fh), jnp.float32),
            pltpu.VMEM_SHARED((NPAD, Fh), jnp.float32),
            pltpu.SemaphoreType.DMA,
        ],
    )
    def prop(xall, src3d, dst2d, out, sidx, didx, rows, acc, sem):
        c = lax.axis_index("c")
        s = lax.axis_index("s")
        r0 = s * ROWS_T
        pltpu.sync_copy(xall.at[pl.ds(c * NPAD + r0, ROWS_T)],
                        acc.at[pl.ds(r0, ROWS_T)])
        plsc.subcore_barrier()

        def blk(bi, carry):
            b0 = s * CPT + bi * BCH
            pltpu.sync_copy(src3d.at[c, pl.ds(b0, BCH)], sidx)
            pltpu.sync_copy(dst2d.at[pl.ds(b0, BCH)], didx)

            def body(j, carry2):
                pltpu.async_copy(xall.at[sidx.at[j]], rows, sem).wait()
                pltpu.sync_copy(rows, acc.at[didx.at[j]], add=True)
                return carry2

            return lax.fori_loop(0, BCH, body, carry)

        lax.fori_loop(0, CPT // BCH, blk, 0)
        plsc.subcore_barrier()
        pltpu.sync_copy(acc.at[pl.ds(r0, ROWS_T)], out.at[c, pl.ds(r0, ROWS_T)])

    return prop


@functools.cache
def _make_prop_es():
    F = 128

    @functools.partial(
        pl.kernel,
        out_type=jax.ShapeDtypeStruct((2, NPAD, F), jnp.float32),
        mesh=_mesh(),
        scratch_types=[
            pltpu.VMEM((BCH, CHUNK), jnp.int32),
            pltpu.VMEM((BCH, CHUNK), jnp.int32),
            pltpu.VMEM((CHUNK, F), jnp.float32),
            pltpu.VMEM_SHARED((NPAD, F), jnp.float32),
            pltpu.SemaphoreType.DMA,
        ],
    )
    def prop(xt, src2d, dst2d, out, sidx, didx, rows, acc, sem):
        c = lax.axis_index("c")
        s = lax.axis_index("s")
        r0 = s * ROWS_T
        pltpu.sync_copy(xt.at[pl.ds(r0, ROWS_T)], acc.at[pl.ds(r0, ROWS_T)])
        e0 = c * (TILES * CPT_DEG) + s * CPT_DEG
        plsc.subcore_barrier()

        def blk(bi, carry):
            b0 = e0 + bi * BCH
            pltpu.sync_copy(src2d.at[pl.ds(b0, BCH)], sidx)
            pltpu.sync_copy(dst2d.at[pl.ds(b0, BCH)], didx)

            def body(j, carry2):
                pltpu.async_copy(xt.at[sidx.at[j]], rows, sem).wait()
                pltpu.sync_copy(rows, acc.at[didx.at[j]], add=True)
                return carry2

            return lax.fori_loop(0, BCH, body, carry)

        lax.fori_loop(0, CPT_DEG // BCH, blk, 0)
        plsc.subcore_barrier()
        pltpu.sync_copy(acc.at[pl.ds(r0, ROWS_T)], out.at[c, pl.ds(r0, ROWS_T)])

    return prop


def _prop_fs(xall, src3d, dstp):
    return _make_prop_fs()(xall, src3d, dstp)


def _prop_es(xt, srcp, dstp):
    return _make_prop_es()(xt, srcp, dstp)


def _row_spec(f):
    return pl.BlockSpec((BR, f), lambda i: (i, 0))


def _full_spec(r, f):
    return pl.BlockSpec((r, f), lambda i: (0, 0))


def _tc0_body(deg_ref, x_ref, a_ref, xs_ref):
    d = deg_ref[:, 0] + deg_ref[:, 1] + 1.0
    a = lax.rsqrt(d)[:, None]
    a_ref[...] = a
    xs_ref[...] = x_ref[...] * a


def _tc0(deg2, x):
    f = x.shape[1]
    return pl.pallas_call(
        _tc0_body,
        grid=(N // BR,),
        in_specs=[pl.BlockSpec((BR, 2), lambda i: (i, 0)), _row_spec(f)],
        out_specs=(_row_spec(1), _row_spec(f)),
        out_shape=(jax.ShapeDtypeStruct((N, 1), jnp.float32),
                   jax.ShapeDtypeStruct((NPAD, f), jnp.float32)),
    )(deg2, x)


def _half_spec(f, which):
    return pl.BlockSpec((1, BR, f), lambda i, _w=which: (_w, i, 0))


def _tcA_body(p_ref, q_ref, t_ref, a_ref, w_ref, b_ref, o_ref, *, fin):
    a = a_ref[...]
    u = ((p_ref[0] + q_ref[0] - t_ref[...]) * a)[:, :fin]
    h = jax.nn.relu(jnp.dot(u, w_ref[...], preferred_element_type=jnp.float32)
                    + b_ref[...])
    o_ref[0] = h * a


def _tcA(pq, t, a, w, b):
    fin = w.shape[0]
    fo = w.shape[1]
    ft = t.shape[1]
    return pl.pallas_call(
        functools.partial(_tcA_body, fin=fin),
        grid=(N // BR, 2),
        in_specs=[pl.BlockSpec((1, BR, ft), lambda i, j: (0, i, 0)),
                  pl.BlockSpec((1, BR, ft), lambda i, j: (1, i, 0)),
                  pl.BlockSpec((BR, ft), lambda i, j: (i, 0)),
                  pl.BlockSpec((BR, 1), lambda i, j: (i, 0)),
                  pl.BlockSpec((fin, fo // 2), lambda i, j: (0, j)),
                  pl.BlockSpec((1, fo // 2), lambda i, j: (0, j))],
        out_specs=pl.BlockSpec((1, BR, fo // 2), lambda i, j: (j, i, 0)),
        out_shape=jax.ShapeDtypeStruct((2, NPAD, fo // 2), jnp.float32),
    )(pq, pq, t, a, w, b.reshape(1, fo))


def _tcB_body(slo_ref, shi_ref, a_ref, w1_ref, b1_ref, w2_ref, o_ref):
    a = a_ref[...]
    u = jnp.concatenate([slo_ref[0], shi_ref[0]], axis=1) * a
    h = jax.nn.relu(jnp.dot(u, w1_ref[...], preferred_element_type=jnp.float32)
                    + b1_ref[...])
    o_ref[...] = jnp.dot(h, w2_ref[...], preferred_element_type=jnp.float32) * a


def _tcB(s, a, w1, b1, w2):
    fi = w1.shape[0]
    fm = w1.shape[1]
    fo = w2.shape[1]
    return pl.pallas_call(
        _tcB_body,
        grid=(N // BR,),
        in_specs=[_half_spec(fi // 2, 0), _half_spec(fi // 2, 1), _row_spec(1),
                  _full_spec(fi, fm), _full_spec(1, fm), _full_spec(fm, fo)],
        out_specs=_row_spec(fo),
        out_shape=jax.ShapeDtypeStruct((NPAD, fo), jnp.float32),
    )(s, s, a, w1, b1.reshape(1, fm), w2)


def _tcC_body(p_ref, q_ref, t_ref, a_ref, be3_ref, wmu_ref, bmu_ref, wlv_ref,
              blv_ref, eps_ref, mu_ref, lv_ref, z_ref):
    a = a_ref[...]
    z2 = (p_ref[0] + q_ref[0] - t_ref[...]) * a + be3_ref[...]
    half = z2.shape[1] // 2
    mu = jnp.dot(z2[:, :half], wmu_ref[...], preferred_element_type=jnp.float32) \
        + bmu_ref[...]
    lv = jnp.dot(z2[:, half:], wlv_ref[...], preferred_element_type=jnp.float32) \
        + blv_ref[...]
    z = (eps_ref[...] * jnp.exp(0.5 * lv) + mu) * a
    mu_ref[...] = mu
    lv_ref[...] = lv
    z_ref[...] = jnp.concatenate([z, jnp.zeros_like(z)], axis=1)


def _tcC(s, t, a, be3, wmu, bmu, wlv, blv, eps):
    dl = wmu.shape[1]
    ft = t.shape[1]
    return pl.pallas_call(
        _tcC_body,
        grid=(N // BR,),
        in_specs=[_half_spec(ft, 0), _half_spec(ft, 1), _row_spec(ft),
                  _row_spec(1),
                  _full_spec(1, 2 * dl), _full_spec(dl, dl), _full_spec(1, dl),
                  _full_spec(dl, dl), _full_spec(1, dl), _row_spec(dl)],
        out_specs=(_row_spec(dl), _row_spec(dl), _row_spec(2 * dl)),
        out_shape=(jax.ShapeDtypeStruct((N, dl), jnp.float32),
                   jax.ShapeDtypeStruct((N, dl), jnp.float32),
                   jax.ShapeDtypeStruct((NPAD, 2 * dl), jnp.float32)),
    )(s, s, t, a, be3.reshape(1, 2 * dl), wmu, bmu.reshape(1, dl),
      wlv, blv.reshape(1, dl), eps)


def _tcD_body(p_ref, q_ref, t_ref, a_ref, b_ref, o_ref):
    o_ref[...] = ((p_ref[0] + q_ref[0] - t_ref[...]) * a_ref[...]
                  + b_ref[...])


def _tcD(s, t, a, b):
    fo = t.shape[1]
    return pl.pallas_call(
        _tcD_body,
        grid=(N // BR,),
        in_specs=[_half_spec(fo, 0), _half_spec(fo, 1), _row_spec(fo),
                  _row_spec(1), _full_spec(1, fo)],
        out_specs=_row_spec(fo),
        out_shape=jax.ShapeDtypeStruct((N, fo), jnp.float32),
    )(s, s, t, a, b.reshape(1, fo))


def kernel(x, edge_index, We1, be1, We2, be2, We3, be3, Wmu, bmu, Wlv, blv,
           Wd1, bd1, Wd2, bd2, Wd3, bd3):
    src = edge_index[0]
    dst = edge_index[1]
    pad = EPAD - E
    srcp = jnp.concatenate(
        [src, jnp.zeros((pad,), src.dtype)]).reshape(EPAD // CHUNK, CHUNK)
    dstp = jnp.concatenate(
        [dst, jnp.full((pad,), DPAD, dst.dtype)]).reshape(EPAD // CHUNK, CHUNK)
    src3d = jnp.stack([srcp, srcp + NPAD])

    deg2 = _deg(dstp).reshape(2, NPAD).T
    eps = jax.random.normal(jax.random.key(42), (N, 64), jnp.float32)

    a, xs = _tc0(deg2, x)
    s1 = _prop_es(xs, srcp, dstp)
    h1 = _tcA(s1, xs, a, We1, be1)
    s2 = _prop_fs(h1.reshape(2 * NPAD, 128), src3d, dstp)
    t3 = _tcB(s2, a, We2, be2, We3)
    s3 = _prop_es(t3, srcp, dstp)
    mu, logvar, zp = _tcC(s3, t3, a, be3, Wmu, bmu, Wlv, blv, eps)
    s4 = _prop_es(zp, srcp, dstp)
    h4 = _tcA(s4, zp, a, Wd1, bd1)
    s5 = _prop_fs(h4.reshape(2 * NPAD, 128), src3d, dstp)
    t6 = _tcB(s5, a, Wd2, bd2, Wd3)
    s6 = _prop_es(t6, srcp, dstp)
    recon = _tcD(s6, t6, a, bd3)
    return (recon, mu, logvar)


# speedup vs baseline: 7.8747x; 1.1576x over previous
import functools

import jax
import jax.numpy as jnp
from jax import lax
from jax.experimental import pallas as pl
from jax.experimental.pallas import tpu as pltpu
from jax.experimental.pallas import tpu_sc as plsc

N = 10000
E = 320000
NPAD = 10240
EPAD = 327680
CHUNK = 128
TILES = 16
CPT = EPAD // TILES // CHUNK
CPT_DEG = EPAD // 2 // TILES // CHUNK
ROWS_T = NPAD // TILES
DPAD = 10016
BR = 2000
BCH = 16


def _mesh():
    return plsc.VectorSubcoreMesh(core_axis_name="c", subcore_axis_name="s")


@functools.cache
def _make_deg():
    @functools.partial(
        pl.kernel,
        out_type=jax.ShapeDtypeStruct((2, TILES, NPAD // TILES), jnp.float32),
        mesh=_mesh(),
        scratch_types=[
            pltpu.VMEM((CPT_DEG, CHUNK), jnp.int32),
            pltpu.VMEM((CHUNK,), jnp.float32),
            pltpu.VMEM((NPAD // TILES,), jnp.float32),
            pltpu.VMEM_SHARED((NPAD,), jnp.float32),
        ],
    )
    def degk(dst2d, out, didx, ones, zeros, acc):
        c = lax.axis_index("c")
        s = lax.axis_index("s")
        for i in range(CHUNK // 16):
            ones[pl.ds(i * 16, 16)] = jnp.full((16,), 1.0, jnp.float32)
        for i in range(NPAD // TILES // 16):
            zeros[pl.ds(i * 16, 16)] = jnp.zeros((16,), jnp.float32)
        pltpu.sync_copy(zeros, acc.at[pl.ds(s * (NPAD // TILES), NPAD // TILES)])
        plsc.subcore_barrier()
        pltpu.sync_copy(
            dst2d.at[pl.ds(c * (TILES * CPT_DEG) + s * CPT_DEG, CPT_DEG)], didx)

        def body(j, carry):
            pltpu.sync_copy(ones, acc.at[didx.at[j]], add=True)
            return carry

        lax.fori_loop(0, CPT_DEG, body, 0)
        plsc.subcore_barrier()
        pltpu.sync_copy(acc.at[pl.ds(s * (NPAD // TILES), NPAD // TILES)],
                        out.at[c, s])

    return degk


def _deg(dst2d):
    return _make_deg()(dst2d)


@functools.cache
def _make_prop_fs():
    Fh = 128

    @functools.partial(
        pl.kernel,
        out_type=jax.ShapeDtypeStruct((2, NPAD, Fh), jnp.float32),
        mesh=_mesh(),
        scratch_types=[
            pltpu.VMEM((BCH, CHUNK), jnp.int32),
            pltpu.VMEM((BCH, CHUNK), jnp.int32),
            pltpu.VMEM((2, CHUNK, Fh), jnp.float32),
            pltpu.VMEM_SHARED((NPAD, Fh), jnp.float32),
            pltpu.SemaphoreType.DMA,
            pltpu.SemaphoreType.DMA,
        ],
    )
    def prop(xall, src3d, dst2d, out, sidx, didx, rows, acc, sem0, sem1):
        c = lax.axis_index("c")
        s = lax.axis_index("s")
        r0 = s * ROWS_T
        pltpu.sync_copy(xall.at[pl.ds(c * NPAD + r0, ROWS_T)],
                        acc.at[pl.ds(r0, ROWS_T)])
        plsc.subcore_barrier()
        sems = (sem0, sem1)

        def blk(bi, carry):
            b0 = s * CPT + bi * BCH
            pltpu.sync_copy(src3d.at[c, pl.ds(b0, BCH)], sidx)
            pltpu.sync_copy(dst2d.at[pl.ds(b0, BCH)], didx)
            h = [None, None]
            h[0] = pltpu.async_copy(xall.at[sidx.at[0]], rows.at[0], sems[0])
            for j in range(BCH):
                if j + 1 < BCH:
                    h[(j + 1) % 2] = pltpu.async_copy(
                        xall.at[sidx.at[j + 1]], rows.at[(j + 1) % 2],
                        sems[(j + 1) % 2])
                h[j % 2].wait()
                pltpu.sync_copy(rows.at[j % 2], acc.at[didx.at[j]], add=True)
            return carry

        lax.fori_loop(0, CPT // BCH, blk, 0)
        plsc.subcore_barrier()
        pltpu.sync_copy(acc.at[pl.ds(r0, ROWS_T)], out.at[c, pl.ds(r0, ROWS_T)])

    return prop


@functools.cache
def _make_prop_es():
    F = 128

    @functools.partial(
        pl.kernel,
        out_type=jax.ShapeDtypeStruct((2, NPAD, F), jnp.float32),
        mesh=_mesh(),
        scratch_types=[
            pltpu.VMEM((BCH, CHUNK), jnp.int32),
            pltpu.VMEM((BCH, CHUNK), jnp.int32),
            pltpu.VMEM((2, CHUNK, F), jnp.float32),
            pltpu.VMEM_SHARED((NPAD, F), jnp.float32),
            pltpu.SemaphoreType.DMA,
            pltpu.SemaphoreType.DMA,
        ],
    )
    def prop(xt, src2d, dst2d, out, sidx, didx, rows, acc, sem0, sem1):
        c = lax.axis_index("c")
        s = lax.axis_index("s")
        r0 = s * ROWS_T
        pltpu.sync_copy(xt.at[pl.ds(r0, ROWS_T)], acc.at[pl.ds(r0, ROWS_T)])
        e0 = c * (TILES * CPT_DEG) + s * CPT_DEG
        plsc.subcore_barrier()
        sems = (sem0, sem1)

        def blk(bi, carry):
            b0 = e0 + bi * BCH
            pltpu.sync_copy(src2d.at[pl.ds(b0, BCH)], sidx)
            pltpu.sync_copy(dst2d.at[pl.ds(b0, BCH)], didx)
            h = [None, None]
            h[0] = pltpu.async_copy(xt.at[sidx.at[0]], rows.at[0], sems[0])
            for j in range(BCH):
                if j + 1 < BCH:
                    h[(j + 1) % 2] = pltpu.async_copy(
                        xt.at[sidx.at[j + 1]], rows.at[(j + 1) % 2],
                        sems[(j + 1) % 2])
                h[j % 2].wait()
                pltpu.sync_copy(rows.at[j % 2], acc.at[didx.at[j]], add=True)
            return carry

        lax.fori_loop(0, CPT_DEG // BCH, blk, 0)
        plsc.subcore_barrier()
        pltpu.sync_copy(acc.at[pl.ds(r0, ROWS_T)], out.at[c, pl.ds(r0, ROWS_T)])

    return prop


def _prop_fs(xall, src3d, dstp):
    return _make_prop_fs()(xall, src3d, dstp)


def _prop_es(xt, srcp, dstp):
    return _make_prop_es()(xt, srcp, dstp)


def _row_spec(f):
    return pl.BlockSpec((BR, f), lambda i: (i, 0))


def _full_spec(r, f):
    return pl.BlockSpec((r, f), lambda i: (0, 0))


def _tc0_body(deg_ref, x_ref, a_ref, xs_ref):
    d = deg_ref[:, 0] + deg_ref[:, 1] + 1.0
    a = lax.rsqrt(d)[:, None]
    a_ref[...] = a
    xs_ref[...] = x_ref[...] * a


def _tc0(deg2, x):
    f = x.shape[1]
    return pl.pallas_call(
        _tc0_body,
        grid=(N // BR,),
        in_specs=[pl.BlockSpec((BR, 2), lambda i: (i, 0)), _row_spec(f)],
        out_specs=(_row_spec(1), _row_spec(f)),
        out_shape=(jax.ShapeDtypeStruct((N, 1), jnp.float32),
                   jax.ShapeDtypeStruct((NPAD, f), jnp.float32)),
    )(deg2, x)


def _half_spec(f, which):
    return pl.BlockSpec((1, BR, f), lambda i, _w=which: (_w, i, 0))


def _tcA_body(p_ref, q_ref, t_ref, a_ref, w_ref, b_ref, o_ref, *, fin):
    a = a_ref[...]
    u = ((p_ref[0] + q_ref[0] - t_ref[...]) * a)[:, :fin]
    h = jax.nn.relu(jnp.dot(u, w_ref[...], preferred_element_type=jnp.float32)
                    + b_ref[...])
    o_ref[0] = h * a


def _tcA(pq, t, a, w, b):
    fin = w.shape[0]
    fo = w.shape[1]
    ft = t.shape[1]
    return pl.pallas_call(
        functools.partial(_tcA_body, fin=fin),
        grid=(N // BR, 2),
        in_specs=[pl.BlockSpec((1, BR, ft), lambda i, j: (0, i, 0)),
                  pl.BlockSpec((1, BR, ft), lambda i, j: (1, i, 0)),
                  pl.BlockSpec((BR, ft), lambda i, j: (i, 0)),
                  pl.BlockSpec((BR, 1), lambda i, j: (i, 0)),
                  pl.BlockSpec((fin, fo // 2), lambda i, j: (0, j)),
                  pl.BlockSpec((1, fo // 2), lambda i, j: (0, j))],
        out_specs=pl.BlockSpec((1, BR, fo // 2), lambda i, j: (j, i, 0)),
        out_shape=jax.ShapeDtypeStruct((2, NPAD, fo // 2), jnp.float32),
    )(pq, pq, t, a, w, b.reshape(1, fo))


def _tcB_body(slo_ref, shi_ref, a_ref, w1_ref, b1_ref, w2_ref, o_ref):
    a = a_ref[...]
    u = jnp.concatenate([slo_ref[0], shi_ref[0]], axis=1) * a
    h = jax.nn.relu(jnp.dot(u, w1_ref[...], preferred_element_type=jnp.float32)
                    + b1_ref[...])
    o_ref[...] = jnp.dot(h, w2_ref[...], preferred_element_type=jnp.float32) * a


def _tcB(s, a, w1, b1, w2):
    fi = w1.shape[0]
    fm = w1.shape[1]
    fo = w2.shape[1]
    return pl.pallas_call(
        _tcB_body,
        grid=(N // BR,),
        in_specs=[_half_spec(fi // 2, 0), _half_spec(fi // 2, 1), _row_spec(1),
                  _full_spec(fi, fm), _full_spec(1, fm), _full_spec(fm, fo)],
        out_specs=_row_spec(fo),
        out_shape=jax.ShapeDtypeStruct((NPAD, fo), jnp.float32),
    )(s, s, a, w1, b1.reshape(1, fm), w2)


def _tcC_body(p_ref, q_ref, t_ref, a_ref, be3_ref, wmu_ref, bmu_ref, wlv_ref,
              blv_ref, eps_ref, mu_ref, lv_ref, z_ref):
    a = a_ref[...]
    z2 = (p_ref[0] + q_ref[0] - t_ref[...]) * a + be3_ref[...]
    half = z2.shape[1] // 2
    mu = jnp.dot(z2[:, :half], wmu_ref[...], preferred_element_type=jnp.float32) \
        + bmu_ref[...]
    lv = jnp.dot(z2[:, half:], wlv_ref[...], preferred_element_type=jnp.float32) \
        + blv_ref[...]
    z = (eps_ref[...] * jnp.exp(0.5 * lv) + mu) * a
    mu_ref[...] = mu
    lv_ref[...] = lv
    z_ref[...] = jnp.concatenate([z, jnp.zeros_like(z)], axis=1)


def _tcC(s, t, a, be3, wmu, bmu, wlv, blv, eps):
    dl = wmu.shape[1]
    ft = t.shape[1]
    return pl.pallas_call(
        _tcC_body,
        grid=(N // BR,),
        in_specs=[_half_spec(ft, 0), _half_spec(ft, 1), _row_spec(ft),
                  _row_spec(1),
                  _full_spec(1, 2 * dl), _full_spec(dl, dl), _full_spec(1, dl),
                  _full_spec(dl, dl), _full_spec(1, dl), _row_spec(dl)],
        out_specs=(_row_spec(dl), _row_spec(dl), _row_spec(2 * dl)),
        out_shape=(jax.ShapeDtypeStruct((N, dl), jnp.float32),
                   jax.ShapeDtypeStruct((N, dl), jnp.float32),
                   jax.ShapeDtypeStruct((NPAD, 2 * dl), jnp.float32)),
    )(s, s, t, a, be3.reshape(1, 2 * dl), wmu, bmu.reshape(1, dl),
      wlv, blv.reshape(1, dl), eps)


def _tcD_body(p_ref, q_ref, t_ref, a_ref, b_ref, o_ref):
    o_ref[...] = ((p_ref[0] + q_ref[0] - t_ref[...]) * a_ref[...]
                  + b_ref[...])


def _tcD(s, t, a, b):
    fo = t.shape[1]
    return pl.pallas_call(
        _tcD_body,
        grid=(N // BR,),
        in_specs=[_half_spec(fo, 0), _half_spec(fo, 1), _row_spec(fo),
                  _row_spec(1), _full_spec(1, fo)],
        out_specs=_row_spec(fo),
        out_shape=jax.ShapeDtypeStruct((N, fo), jnp.float32),
    )(s, s, t, a, b.reshape(1, fo))


def kernel(x, edge_index, We1, be1, We2, be2, We3, be3, Wmu, bmu, Wlv, blv,
           Wd1, bd1, Wd2, bd2, Wd3, bd3):
    src = edge_index[0]
    dst = edge_index[1]
    pad = EPAD - E
    srcp = jnp.concatenate(
        [src, jnp.zeros((pad,), src.dtype)]).reshape(EPAD // CHUNK, CHUNK)
    dstp = jnp.concatenate(
        [dst, jnp.full((pad,), DPAD, dst.dtype)]).reshape(EPAD // CHUNK, CHUNK)
    src3d = jnp.stack([srcp, srcp + NPAD])

    deg2 = _deg(dstp).reshape(2, NPAD).T
    eps = jax.random.normal(jax.random.key(42), (N, 64), jnp.float32)

    a, xs = _tc0(deg2, x)
    s1 = _prop_es(xs, srcp, dstp)
    h1 = _tcA(s1, xs, a, We1, be1)
    s2 = _prop_fs(h1.reshape(2 * NPAD, 128), src3d, dstp)
    t3 = _tcB(s2, a, We2, be2, We3)
    s3 = _prop_es(t3, srcp, dstp)
    mu, logvar, zp = _tcC(s3, t3, a, be3, Wmu, bmu, Wlv, blv, eps)
    s4 = _prop_es(zp, srcp, dstp)
    h4 = _tcA(s4, zp, a, Wd1, bd1)
    s5 = _prop_fs(h4.reshape(2 * NPAD, 128), src3d, dstp)
    t6 = _tcB(s5, a, Wd2, bd2, Wd3)
    s6 = _prop_es(t6, srcp, dstp)
    recon = _tcD(s6, t6, a, bd3)
    return (recon, mu, logvar)


# speedup vs baseline: 7.8830x; 1.0010x over previous
import functools

import jax
import jax.numpy as jnp
from jax import lax
from jax.experimental import pallas as pl
from jax.experimental.pallas import tpu as pltpu
from jax.experimental.pallas import tpu_sc as plsc

N = 10000
E = 320000
NPAD = 10240
EPAD = 327680
CHUNK = 128
TILES = 16
CPT = EPAD // TILES // CHUNK
CPT_DEG = EPAD // 2 // TILES // CHUNK
ROWS_T = NPAD // TILES
DPAD = 10016
BR = 2000
BCH = 16


def _mesh():
    return plsc.VectorSubcoreMesh(core_axis_name="c", subcore_axis_name="s")


@functools.cache
def _make_deg():
    @functools.partial(
        pl.kernel,
        out_type=jax.ShapeDtypeStruct((2, TILES, NPAD // TILES), jnp.float32),
        mesh=_mesh(),
        scratch_types=[
            pltpu.VMEM((CPT_DEG, CHUNK), jnp.int32),
            pltpu.VMEM((CHUNK,), jnp.float32),
            pltpu.VMEM((NPAD // TILES,), jnp.float32),
            pltpu.VMEM_SHARED((NPAD,), jnp.float32),
        ],
    )
    def degk(dst2d, out, didx, ones, zeros, acc):
        c = lax.axis_index("c")
        s = lax.axis_index("s")
        for i in range(CHUNK // 16):
            ones[pl.ds(i * 16, 16)] = jnp.full((16,), 1.0, jnp.float32)
        for i in range(NPAD // TILES // 16):
            zeros[pl.ds(i * 16, 16)] = jnp.zeros((16,), jnp.float32)
        pltpu.sync_copy(zeros, acc.at[pl.ds(s * (NPAD // TILES), NPAD // TILES)])
        plsc.subcore_barrier()
        pltpu.sync_copy(
            dst2d.at[pl.ds(c * (TILES * CPT_DEG) + s * CPT_DEG, CPT_DEG)], didx)

        def body(j, carry):
            pltpu.sync_copy(ones, acc.at[didx.at[j]], add=True)
            return carry

        lax.fori_loop(0, CPT_DEG, body, 0)
        plsc.subcore_barrier()
        pltpu.sync_copy(acc.at[pl.ds(s * (NPAD // TILES), NPAD // TILES)],
                        out.at[c, s])

    return degk


def _deg(dst2d):
    return _make_deg()(dst2d)


@functools.cache
def _make_prop_fs():
    Fh = 128

    @functools.partial(
        pl.kernel,
        out_type=jax.ShapeDtypeStruct((2, NPAD, Fh), jnp.float32),
        mesh=_mesh(),
        scratch_types=[
            pltpu.VMEM((BCH, CHUNK), jnp.int32),
            pltpu.VMEM((BCH, CHUNK), jnp.int32),
            pltpu.VMEM((2, CHUNK, Fh), jnp.float32),
            pltpu.VMEM_SHARED((NPAD, Fh), jnp.float32),
            pltpu.SemaphoreType.DMA,
            pltpu.SemaphoreType.DMA,
        ],
    )
    def prop(xall, src3d, dst2d, out, sidx, didx, rows, acc, sem0, sem1):
        c = lax.axis_index("c")
        s = lax.axis_index("s")
        r0 = s * ROWS_T
        pltpu.sync_copy(xall.at[pl.ds(c * NPAD + r0, ROWS_T)],
                        acc.at[pl.ds(r0, ROWS_T)])
        plsc.subcore_barrier()
        sems = (sem0, sem1)

        def blk(bi, carry):
            b0 = s * CPT + bi * BCH
            pltpu.sync_copy(src3d.at[c, pl.ds(b0, BCH)], sidx)
            pltpu.sync_copy(dst2d.at[pl.ds(b0, BCH)], didx)
            h = [None, None]
            h[0] = pltpu.async_copy(xall.at[sidx.at[0]], rows.at[0], sems[0])
            for j in range(BCH):
                if j + 1 < BCH:
                    h[(j + 1) % 2] = pltpu.async_copy(
                        xall.at[sidx.at[j + 1]], rows.at[(j + 1) % 2],
                        sems[(j + 1) % 2])
                h[j % 2].wait()
                pltpu.sync_copy(rows.at[j % 2], acc.at[didx.at[j]], add=True)
            return carry

        lax.fori_loop(0, CPT // BCH, blk, 0)
        plsc.subcore_barrier()
        pltpu.sync_copy(acc.at[pl.ds(r0, ROWS_T)], out.at[c, pl.ds(r0, ROWS_T)])

    return prop


@functools.cache
def _make_prop_es(F):
    @functools.partial(
        pl.kernel,
        out_type=jax.ShapeDtypeStruct((2, NPAD, F), jnp.float32),
        mesh=_mesh(),
        scratch_types=[
            pltpu.VMEM((BCH, CHUNK), jnp.int32),
            pltpu.VMEM((BCH, CHUNK), jnp.int32),
            pltpu.VMEM((2, CHUNK, F), jnp.float32),
            pltpu.VMEM_SHARED((NPAD, F), jnp.float32),
            pltpu.SemaphoreType.DMA,
            pltpu.SemaphoreType.DMA,
        ],
    )
    def prop(xt, src2d, dst2d, out, sidx, didx, rows, acc, sem0, sem1):
        c = lax.axis_index("c")
        s = lax.axis_index("s")
        r0 = s * ROWS_T
        pltpu.sync_copy(xt.at[pl.ds(r0, ROWS_T)], acc.at[pl.ds(r0, ROWS_T)])
        e0 = c * (TILES * CPT_DEG) + s * CPT_DEG
        plsc.subcore_barrier()
        sems = (sem0, sem1)

        def blk(bi, carry):
            b0 = e0 + bi * BCH
            pltpu.sync_copy(src2d.at[pl.ds(b0, BCH)], sidx)
            pltpu.sync_copy(dst2d.at[pl.ds(b0, BCH)], didx)
            h = [None, None]
            h[0] = pltpu.async_copy(xt.at[sidx.at[0]], rows.at[0], sems[0])
            for j in range(BCH):
                if j + 1 < BCH:
                    h[(j + 1) % 2] = pltpu.async_copy(
                        xt.at[sidx.at[j + 1]], rows.at[(j + 1) % 2],
                        sems[(j + 1) % 2])
                h[j % 2].wait()
                pltpu.sync_copy(rows.at[j % 2], acc.at[didx.at[j]], add=True)
            return carry

        lax.fori_loop(0, CPT_DEG // BCH, blk, 0)
        plsc.subcore_barrier()
        pltpu.sync_copy(acc.at[pl.ds(r0, ROWS_T)], out.at[c, pl.ds(r0, ROWS_T)])

    return prop


def _prop_fs(xall, src3d, dstp):
    return _make_prop_fs()(xall, src3d, dstp)


def _prop_es(xt, srcp, dstp):
    return _make_prop_es(xt.shape[1])(xt, srcp, dstp)


def _row_spec(f):
    return pl.BlockSpec((BR, f), lambda i: (i, 0))


def _full_spec(r, f):
    return pl.BlockSpec((r, f), lambda i: (0, 0))


def _tc0_body(deg_ref, x_ref, a_ref, xs_ref):
    d = deg_ref[:, 0] + deg_ref[:, 1] + 1.0
    a = lax.rsqrt(d)[:, None]
    a_ref[...] = a
    xs_ref[...] = x_ref[...] * a


def _tc0(deg2, x):
    f = x.shape[1]
    return pl.pallas_call(
        _tc0_body,
        grid=(N // BR,),
        in_specs=[pl.BlockSpec((BR, 2), lambda i: (i, 0)), _row_spec(f)],
        out_specs=(_row_spec(1), _row_spec(f)),
        out_shape=(jax.ShapeDtypeStruct((N, 1), jnp.float32),
                   jax.ShapeDtypeStruct((NPAD, f), jnp.float32)),
    )(deg2, x)


def _half_spec(f, which):
    return pl.BlockSpec((1, BR, f), lambda i, _w=which: (_w, i, 0))


def _tcA_body(p_ref, q_ref, t_ref, a_ref, w_ref, b_ref, o_ref, *, fin):
    a = a_ref[...]
    u = ((p_ref[0] + q_ref[0] - t_ref[...]) * a)[:, :fin]
    h = jax.nn.relu(jnp.dot(u, w_ref[...], preferred_element_type=jnp.float32)
                    + b_ref[...])
    o_ref[0] = h * a


def _tcA(pq, t, a, w, b):
    fin = w.shape[0]
    fo = w.shape[1]
    ft = t.shape[1]
    return pl.pallas_call(
        functools.partial(_tcA_body, fin=fin),
        grid=(N // BR, 2),
        in_specs=[pl.BlockSpec((1, BR, ft), lambda i, j: (0, i, 0)),
                  pl.BlockSpec((1, BR, ft), lambda i, j: (1, i, 0)),
                  pl.BlockSpec((BR, ft), lambda i, j: (i, 0)),
                  pl.BlockSpec((BR, 1), lambda i, j: (i, 0)),
                  pl.BlockSpec((fin, fo // 2), lambda i, j: (0, j)),
                  pl.BlockSpec((1, fo // 2), lambda i, j: (0, j))],
        out_specs=pl.BlockSpec((1, BR, fo // 2), lambda i, j: (j, i, 0)),
        out_shape=jax.ShapeDtypeStruct((2, NPAD, fo // 2), jnp.float32),
    )(pq, pq, t, a, w, b.reshape(1, fo))


def _tcB_body(slo_ref, shi_ref, a_ref, w1_ref, b1_ref, w2_ref, o_ref):
    a = a_ref[...]
    u = jnp.concatenate([slo_ref[0], shi_ref[0]], axis=1) * a
    h = jax.nn.relu(jnp.dot(u, w1_ref[...], preferred_element_type=jnp.float32)
                    + b1_ref[...])
    o_ref[...] = jnp.dot(h, w2_ref[...], preferred_element_type=jnp.float32) * a


def _tcB(s, a, w1, b1, w2):
    fi = w1.shape[0]
    fm = w1.shape[1]
    fo = w2.shape[1]
    return pl.pallas_call(
        _tcB_body,
        grid=(N // BR,),
        in_specs=[_half_spec(fi // 2, 0), _half_spec(fi // 2, 1), _row_spec(1),
                  _full_spec(fi, fm), _full_spec(1, fm), _full_spec(fm, fo)],
        out_specs=_row_spec(fo),
        out_shape=jax.ShapeDtypeStruct((NPAD, fo), jnp.float32),
    )(s, s, a, w1, b1.reshape(1, fm), w2)


def _tcC_body(p_ref, q_ref, t_ref, a_ref, be3_ref, wmu_ref, bmu_ref, wlv_ref,
              blv_ref, eps_ref, mu_ref, lv_ref, z_ref):
    a = a_ref[...]
    z2 = (p_ref[0] + q_ref[0] - t_ref[...]) * a + be3_ref[...]
    half = z2.shape[1] // 2
    mu = jnp.dot(z2[:, :half], wmu_ref[...], preferred_element_type=jnp.float32) \
        + bmu_ref[...]
    lv = jnp.dot(z2[:, half:], wlv_ref[...], preferred_element_type=jnp.float32) \
        + blv_ref[...]
    z = (eps_ref[...] * jnp.exp(0.5 * lv) + mu) * a
    mu_ref[...] = mu
    lv_ref[...] = lv
    z_ref[...] = jnp.concatenate([z, jnp.zeros_like(z)], axis=1)


def _tcC(s, t, a, be3, wmu, bmu, wlv, blv, eps):
    dl = wmu.shape[1]
    ft = t.shape[1]
    return pl.pallas_call(
        _tcC_body,
        grid=(N // BR,),
        in_specs=[_half_spec(ft, 0), _half_spec(ft, 1), _row_spec(ft),
                  _row_spec(1),
                  _full_spec(1, 2 * dl), _full_spec(dl, dl), _full_spec(1, dl),
                  _full_spec(dl, dl), _full_spec(1, dl), _row_spec(dl)],
        out_specs=(_row_spec(dl), _row_spec(dl), _row_spec(2 * dl)),
        out_shape=(jax.ShapeDtypeStruct((N, dl), jnp.float32),
                   jax.ShapeDtypeStruct((N, dl), jnp.float32),
                   jax.ShapeDtypeStruct((NPAD, 2 * dl), jnp.float32)),
    )(s, s, t, a, be3.reshape(1, 2 * dl), wmu, bmu.reshape(1, dl),
      wlv, blv.reshape(1, dl), eps)


def _tcD_body(p_ref, q_ref, t_ref, a_ref, b_ref, o_ref):
    o_ref[...] = ((p_ref[0] + q_ref[0] - t_ref[...]) * a_ref[...]
                  + b_ref[...])


def _tcD(s, t, a, b):
    fo = t.shape[1]
    return pl.pallas_call(
        _tcD_body,
        grid=(N // BR,),
        in_specs=[_half_spec(fo, 0), _half_spec(fo, 1), _row_spec(fo),
                  _row_spec(1), _full_spec(1, fo)],
        out_specs=_row_spec(fo),
        out_shape=jax.ShapeDtypeStruct((N, fo), jnp.float32),
    )(s, s, t, a, b.reshape(1, fo))


def kernel(x, edge_index, We1, be1, We2, be2, We3, be3, Wmu, bmu, Wlv, blv,
           Wd1, bd1, Wd2, bd2, Wd3, bd3):
    src = edge_index[0]
    dst = edge_index[1]
    pad = EPAD - E
    srcp = jnp.concatenate(
        [src, jnp.zeros((pad,), src.dtype)]).reshape(EPAD // CHUNK, CHUNK)
    dstp = jnp.concatenate(
        [dst, jnp.full((pad,), DPAD, dst.dtype)]).reshape(EPAD // CHUNK, CHUNK)
    src3d = jnp.stack([srcp, srcp + NPAD])

    deg2 = _deg(dstp).reshape(2, NPAD).T
    eps = jax.random.normal(jax.random.key(42), (N, 64), jnp.float32)

    a, xs = _tc0(deg2, x)
    s1 = _prop_es(xs, srcp, dstp)
    h1 = _tcA(s1, xs, a, We1, be1)
    s2 = _prop_fs(h1.reshape(2 * NPAD, 128), src3d, dstp)
    t3 = _tcB(s2, a, We2, be2, We3)
    s3 = _prop_es(t3, srcp, dstp)
    mu, logvar, zp = _tcC(s3, t3, a, be3, Wmu, bmu, Wlv, blv, eps)
    s4 = _prop_es(zp, srcp, dstp)
    h4 = _tcA(s4, zp, a, Wd1, bd1)
    s5 = _prop_fs(h4.reshape(2 * NPAD, 128), src3d, dstp)
    t6 = _tcB(s5, a, Wd2, bd2, Wd3)
    s6 = _prop_es(t6, srcp, dstp)
    recon = _tcD(s6, t6, a, bd3)
    return (recon, mu, logvar)
